# baseline (speedup 1.0000x reference)
import functools

import jax
import jax.numpy as jnp
from jax.experimental import pallas as pl


_ITERS = 9


def _body(x_ref, o_ref, *, iters):
    x = x_ref[...]
    n = x.shape[1]
    rowsum = jnp.sum(x, axis=1, keepdims=True)
    xmax = jnp.max(x, axis=1, keepdims=True)
    alpha = 1.0 / jnp.abs(rowsum)
    z = x * alpha
    o_ref[...] = z
    zmax = xmax * alpha

    lo = zmax - 1.0
    hi = zmax
    sgn = jnp.sign(rowsum)
    t = jnp.maximum(lo, (sgn - 1.0) / n)
    tau = jnp.full_like(zmax, -3e38)
    for _ in range(iters):
        zz = o_ref[...]
        r = jnp.maximum(zz - t, 0.0)
        f = jnp.sum(r, axis=1, keepdims=True)
        k = jnp.sum(jnp.sign(r), axis=1, keepdims=True)
        k = jnp.maximum(k, 1.0)
        t_n = t + (f - 1.0) / k
        ge = f >= 1.0
        lo = jnp.where(ge, t, lo)
        hi = jnp.where(ge, hi, t)
        tau = jnp.maximum(tau, t_n)
        t = jnp.maximum(t_n, 0.5 * (lo + hi))

    o_ref[...] = jnp.maximum(o_ref[...] - tau, 0.0)


def kernel(input):
    bs, dim = input.shape
    x = input.astype(jnp.float32)
    rows_per_block = 8
    grid = (bs // rows_per_block,)
    out = pl.pallas_call(
        functools.partial(_body, iters=_ITERS),
        grid=grid,
        in_specs=[pl.BlockSpec((rows_per_block, dim), lambda i: (i, 0))],
        out_specs=pl.BlockSpec((rows_per_block, dim), lambda i: (i, 0)),
        out_shape=jax.ShapeDtypeStruct((bs, dim), jnp.float32),
    )(x)
    return out


# speedup vs baseline: 1.2705x; 1.2705x over previous
import functools

import jax
import jax.numpy as jnp
from jax.experimental import pallas as pl


_ITERS = 9


def _body(x_ref, o_ref, *, iters):
    x = x_ref[...]
    n = x.shape[1]
    rowsum = jnp.sum(x, axis=1, keepdims=True)
    xmax = jnp.max(x, axis=1, keepdims=True)
    alpha = 1.0 / jnp.abs(rowsum)
    z = x * alpha
    o_ref[...] = z
    zmax = xmax * alpha

    lo = zmax - 1.0
    hi = zmax
    sgn = jnp.sign(rowsum)
    t = jnp.maximum(lo, (sgn - 1.0) / n)
    tau = jnp.full_like(zmax, -3e38)
    for _ in range(iters):
        zz = o_ref[...]
        d = zz - t
        f = jnp.sum(jnp.maximum(d, 0.0), axis=1, keepdims=True)
        k = jnp.sum(jnp.where(d > 0.0, 1.0, 0.0), axis=1, keepdims=True)
        k = jnp.maximum(k, 1.0)
        t_n = t + (f - 1.0) / k
        ge = f >= 1.0
        lo = jnp.where(ge, t, lo)
        hi = jnp.where(ge, hi, t)
        tau = jnp.maximum(tau, t_n)
        t = jnp.maximum(t_n, 0.5 * (lo + hi))

    o_ref[...] = jnp.maximum(o_ref[...] - tau, 0.0)


def kernel(input):
    bs, dim = input.shape
    x = input.astype(jnp.float32)
    rows_per_block = 8
    grid = (bs // rows_per_block,)
    out = pl.pallas_call(
        functools.partial(_body, iters=_ITERS),
        grid=grid,
        in_specs=[pl.BlockSpec((rows_per_block, dim), lambda i: (i, 0))],
        out_specs=pl.BlockSpec((rows_per_block, dim), lambda i: (i, 0)),
        out_shape=jax.ShapeDtypeStruct((bs, dim), jnp.float32),
    )(x)
    return out
